# baseline (speedup 1.0000x reference)
import jax
import jax.numpy as jnp
from jax import lax
from jax.experimental import pallas as pl
from jax.experimental.pallas import tpu as pltpu
from jax.experimental.pallas import tpu_sc as plsc

VOCAB = 1_000_000
D = 32
NTOK = 819_200

_NC = 2
_NS = 16
_NW = _NC * _NS
_BPW = NTOK // _NW
_C = 128
_K = 5
_KC = _K * _C
_NSUP = _BPW // _KC


def _emb_body(idx_hbm, table_hbm, out_hbm, idx_v, bufa, bufb, tba, tbb,
              gsa, gsb, osa, osb):
    wid = lax.axis_index("s") * _NC + lax.axis_index("c")
    base = wid * _BPW
    pltpu.sync_copy(idx_hbm.at[pl.ds(base, _BPW)], idx_v)

    iota = lax.iota(jnp.int32, 16)

    def fire(buf, sem, s):
        for k in range(_K):
            ic = idx_v.at[pl.ds(s * _KC + k * _C, _C)]
            pltpu.async_copy(table_hbm.at[ic], buf.at[pl.ds(k * _C, _C)], sem)

    def drain_g(buf, sem):
        pltpu.make_async_copy(table_hbm.at[pl.ds(0, _KC)], buf, sem).wait()

    def transpose(buf, tbuf):
        def col(c, carry):
            rows = iota + c * 16
            for f in range(D):
                v = plsc.load_gather(buf, [rows, jnp.full((16,), f, jnp.int32)])
                tbuf[f, pl.ds(c * 16, 16)] = v
            return carry
        lax.fori_loop(0, _KC // 16, col, 0, unroll=False)

    def start_o(tbuf, sem, s):
        pltpu.async_copy(tbuf, out_hbm.at[:, pl.ds(base + s * _KC, _KC)], sem)

    def wait_o(tbuf, sem):
        pltpu.make_async_copy(tbuf, out_hbm.at[:, pl.ds(base, _KC)], sem).wait()

    fire(bufa, gsa, 0)

    def body(i2, carry):
        s = i2 * 2
        fire(bufb, gsb, s + 1)
        drain_g(bufa, gsa)
        pl.when(s >= 2)(lambda: wait_o(tba, osa))
        transpose(bufa, tba)
        start_o(tba, osa, s)
        pl.when(s + 2 < _NSUP)(lambda: fire(bufa, gsa, s + 2))
        drain_g(bufb, gsb)
        pl.when(s >= 2)(lambda: wait_o(tbb, osb))
        transpose(bufb, tbb)
        start_o(tbb, osb, s + 1)
        return carry

    lax.fori_loop(0, _NSUP // 2, body, 0)
    wait_o(tba, osa)
    wait_o(tbb, osb)


_emb = pl.kernel(
    _emb_body,
    out_type=jax.ShapeDtypeStruct((D, NTOK), jnp.float32),
    mesh=plsc.VectorSubcoreMesh(core_axis_name="c", subcore_axis_name="s"),
    scratch_types=[
        pltpu.VMEM((_BPW,), jnp.int32),
        pltpu.VMEM((_KC, D), jnp.float32),
        pltpu.VMEM((_KC, D), jnp.float32),
        pltpu.VMEM((D, _KC), jnp.float32),
        pltpu.VMEM((D, _KC), jnp.float32),
        pltpu.SemaphoreType.DMA,
        pltpu.SemaphoreType.DMA,
        pltpu.SemaphoreType.DMA,
        pltpu.SemaphoreType.DMA,
    ],
    compiler_params=pltpu.CompilerParams(
        use_tc_tiling_on_sc=False, needs_layout_passes=False
    ),
)


@jax.jit
def kernel(x, weights):
    return _emb(x.astype(jnp.int32), weights).T


# speedup vs baseline: 4.3699x; 4.3699x over previous
import jax
import jax.numpy as jnp
from jax import lax
from jax.experimental import pallas as pl
from jax.experimental.pallas import tpu as pltpu
from jax.experimental.pallas import tpu_sc as plsc

VOCAB = 1_000_000
D = 32
NTOK = 819_200

_NC = 2
_NS = 16
_NW = _NC * _NS
_BPW = NTOK // _NW
_C = 256
_NCH = _BPW // _C
_PR = VOCAB // 4


def _emb_body(idx_hbm, table_hbm, out_hbm, idx_v, pidx_v, rows, tbuf,
              gsa, gsb, osa, osb):
    wid = lax.axis_index("s") * _NC + lax.axis_index("c")
    base = wid * _BPW
    pltpu.sync_copy(idx_hbm.at[pl.ds(base, _BPW)], idx_v)

    iota = lax.iota(jnp.int32, 16)
    gsem = [gsa, gsb]
    osem = [osa, osb]

    def make_pidx(h, j):
        @plsc.parallel_loop(0, _C // 16, 1, unroll=2)
        def _(g):
            v = idx_v[pl.ds(j * _C + g * 16, 16)]
            pidx_v[pl.ds(h * _C + g * 16, 16)] = lax.shift_right_logical(v, 2)

    def fire(h, j):
        del j
        for k in range(_C // 128):
            pltpu.async_copy(
                table_hbm.at[pidx_v.at[pl.ds(h * _C + k * 128, 128)]],
                rows.at[h, pl.ds(k * 128, 128)],
                gsem[h],
            )

    def drain_g(h):
        pltpu.make_async_copy(
            table_hbm.at[pl.ds(0, _C)], rows.at[h], gsem[h]
        ).wait()

    def extract(h, j):
        @plsc.parallel_loop(0, _C // 16, 1, unroll=2)
        def _(g):
            u0 = g * 16
            idxv = idx_v[pl.ds(j * _C + u0, 16)]
            qcol = lax.mul(lax.bitwise_and(idxv, 3), 32)
            rowi = iota + u0
            for f in range(D):
                v = plsc.load_gather(rows.at[h], [rowi, qcol + f])
                tbuf[h, f // 8, f % 8, pl.ds(u0, 16)] = v

    def start_o(h, j):
        tok0 = base + j * _C
        for t in range(4):
            pltpu.async_copy(
                tbuf.at[h, t],
                out_hbm.at[pl.ds(8 * t, 8), pl.ds(tok0, _C)],
                osem[h],
            )

    def wait_o(h):
        for t in range(4):
            pltpu.make_async_copy(
                tbuf.at[h, t], out_hbm.at[pl.ds(0, 8), pl.ds(0, _C)], osem[h]
            ).wait()

    make_pidx(0, 0)
    fire(0, 0)

    def body(i2, carry):
        j = i2 * 2
        for h in (0, 1):
            jj = j + h
            def prefetch(jn=jj + 1, hn=1 - h):
                make_pidx(hn, jn)
                fire(hn, jn)
            pl.when(jj + 1 < _NCH)(prefetch)
            drain_g(h)
            pl.when(jj >= 2)(lambda hh=h: wait_o(hh))
            extract(h, jj)
            start_o(h, jj)
        return carry

    lax.fori_loop(0, _NCH // 2, body, 0)
    wait_o(0)
    wait_o(1)


_emb = pl.kernel(
    _emb_body,
    out_type=jax.ShapeDtypeStruct((D, NTOK), jnp.float32),
    mesh=plsc.VectorSubcoreMesh(core_axis_name="c", subcore_axis_name="s"),
    scratch_types=[
        pltpu.VMEM((_BPW,), jnp.int32),
        pltpu.VMEM((2 * _C,), jnp.int32),
        pltpu.VMEM((2, _C, 128), jnp.float32),
        pltpu.VMEM((2, 4, 8, _C), jnp.float32),
        pltpu.SemaphoreType.DMA,
        pltpu.SemaphoreType.DMA,
        pltpu.SemaphoreType.DMA,
        pltpu.SemaphoreType.DMA,
    ],
    compiler_params=pltpu.CompilerParams(
        use_tc_tiling_on_sc=True, needs_layout_passes=False
    ),
)


@jax.jit
def kernel(x, weights):
    w128 = weights.reshape(_PR, 128)
    return _emb(x.astype(jnp.int32), w128).T
